# Initial kernel scaffold; baseline (speedup 1.0000x reference)
import functools

import jax
import jax.numpy as jnp
from jax import lax
from jax.experimental import pallas as pl
from jax.experimental.pallas import tpu as pltpu
from jax.experimental.pallas import tpu_sc as plsc

EMBED_DIM = 32

_NC = 2
_NS = 16
_NW = _NC * _NS

_CHUNK = 2048


def _emb_body(n_total, x_hbm, table_hbm, out_hbm, idx_v, rows_v, sem):
    wid = lax.axis_index("s") * _NC + lax.axis_index("c")
    per_w = n_total // _NW
    n_chunks = per_w // _CHUNK

    def body(i, carry):
        base = wid * per_w + i * _CHUNK
        pltpu.sync_copy(x_hbm.at[pl.ds(base, _CHUNK)], idx_v)
        pltpu.async_copy(table_hbm.at[idx_v], rows_v, sem).wait()
        pltpu.sync_copy(rows_v, out_hbm.at[pl.ds(base, _CHUNK)])
        return carry

    lax.fori_loop(0, n_chunks, body, 0)


def kernel(x, table):
    batch, hist = x.shape
    n_total = batch * hist
    flat = x.reshape(n_total)

    mesh = plsc.VectorSubcoreMesh(core_axis_name="c", subcore_axis_name="s")
    out = pl.kernel(
        functools.partial(_emb_body, n_total),
        mesh=mesh,
        out_type=jax.ShapeDtypeStruct((n_total, EMBED_DIM), jnp.float32),
        scratch_types=[
            pltpu.VMEM((_CHUNK,), jnp.int32),
            pltpu.VMEM((_CHUNK, EMBED_DIM), jnp.float32),
            pltpu.SemaphoreType.DMA,
        ],
    )(flat, table)
    return out.reshape(batch, hist, EMBED_DIM)


# speedup vs baseline: 6.3423x; 6.3423x over previous
import functools

import jax
import jax.numpy as jnp
from jax import lax
from jax.experimental import pallas as pl
from jax.experimental.pallas import tpu as pltpu
from jax.experimental.pallas import tpu_sc as plsc

EMBED_DIM = 32

_NC = 2
_NS = 16
_NW = _NC * _NS

_CHUNK = 2048


def _emb_body(n_total, x_hbm, table_hbm, out_hbm, idx_v, rows_v, sem):
    wid = lax.axis_index("s") * _NC + lax.axis_index("c")
    per_w = n_total // _NW
    n_chunks = per_w // _CHUNK

    def body(i, carry):
        base = wid * per_w + i * _CHUNK
        pltpu.sync_copy(x_hbm.at[pl.ds(base, _CHUNK)], idx_v)
        pltpu.async_copy(table_hbm.at[idx_v], rows_v, sem).wait()
        pltpu.sync_copy(rows_v, out_hbm.at[pl.ds(base, _CHUNK)])
        return carry

    lax.fori_loop(0, n_chunks, body, 0)


def kernel(x, table):
    batch, hist = x.shape
    n_total = batch * hist
    flat = x.reshape(n_total)

    mesh = plsc.VectorSubcoreMesh(core_axis_name="c", subcore_axis_name="s")
    out = pl.kernel(
        functools.partial(_emb_body, n_total),
        mesh=mesh,
        compiler_params=pltpu.CompilerParams(use_tc_tiling_on_sc=False),
        out_type=jax.ShapeDtypeStruct((n_total, EMBED_DIM), jnp.float32),
        scratch_types=[
            pltpu.VMEM((_CHUNK,), jnp.int32),
            pltpu.VMEM((_CHUNK, EMBED_DIM), jnp.float32),
            pltpu.SemaphoreType.DMA,
        ],
    )(flat, table)
    return out.reshape(batch, hist, EMBED_DIM)


# speedup vs baseline: 6.3721x; 1.0047x over previous
import functools

import jax
import jax.numpy as jnp
from jax import lax
from jax.experimental import pallas as pl
from jax.experimental.pallas import tpu as pltpu
from jax.experimental.pallas import tpu_sc as plsc

EMBED_DIM = 32

_NC = 2
_NS = 16
_NW = _NC * _NS

_CHUNK = 800
_NBUF = 4


def _emb_body(n_total, x_hbm, table_hbm, out_hbm, idx_v, rows_v, gsem, ssem):
    wid = lax.axis_index("s") * _NC + lax.axis_index("c")
    per_w = n_total // _NW
    n_chunks = per_w // _CHUNK
    n_outer = n_chunks // _NBUF
    base0 = wid * per_w

    def stage_and_gather(i, b):
        pltpu.sync_copy(x_hbm.at[pl.ds(base0 + i * _CHUNK, _CHUNK)],
                        idx_v.at[b])
        pltpu.async_copy(table_hbm.at[idx_v.at[b]], rows_v.at[b], gsem)

    def wait_gather(b):
        pltpu.make_async_copy(table_hbm.at[idx_v.at[b]], rows_v.at[b],
                              gsem).wait()

    def scatter(i, b):
        pltpu.async_copy(rows_v.at[b],
                         out_hbm.at[pl.ds(base0 + i * _CHUNK, _CHUNK)], ssem)

    def wait_one_scatter(b):
        pltpu.make_async_copy(
            rows_v.at[b], out_hbm.at[pl.ds(base0, _CHUNK)], ssem).wait()

    for b in range(_NBUF):
        stage_and_gather(b, b)

    def outer(g, carry):
        for b in range(_NBUF):
            i = g * _NBUF + b
            wait_gather(b)
            scatter(i, b)
            wait_one_scatter(b)
            stage_and_gather(i + _NBUF, b)
        return carry

    lax.fori_loop(0, n_outer - 1, outer, 0)

    for b in range(_NBUF):
        i = n_chunks - _NBUF + b
        wait_gather(b)
        scatter(i, b)
    for b in range(_NBUF):
        wait_one_scatter(b)


def kernel(x, table):
    batch, hist = x.shape
    n_total = batch * hist
    flat = x.reshape(n_total)

    mesh = plsc.VectorSubcoreMesh(core_axis_name="c", subcore_axis_name="s")
    out = pl.kernel(
        functools.partial(_emb_body, n_total),
        mesh=mesh,
        compiler_params=pltpu.CompilerParams(use_tc_tiling_on_sc=False),
        out_type=jax.ShapeDtypeStruct((n_total, EMBED_DIM), jnp.float32),
        scratch_types=[
            pltpu.VMEM((_NBUF, _CHUNK), jnp.int32),
            pltpu.VMEM((_NBUF, _CHUNK, EMBED_DIM), jnp.float32),
            pltpu.SemaphoreType.DMA,
            pltpu.SemaphoreType.DMA,
        ],
    )(flat, table)
    return out.reshape(batch, hist, EMBED_DIM)
